# baseline (speedup 1.0000x reference)
import functools

import jax
import jax.numpy as jnp
from jax import lax
from jax.experimental import pallas as pl
from jax.experimental.pallas import tpu as pltpu
from jax.experimental.pallas import tpu_sc as plsc


_LOG2E = 1.4426950408889634
_LN2 = 0.6931471805599453


def _softplus(v):
    return jnp.log2(1.0 + jnp.exp2(v * _LOG2E)) * _LN2


def _emb_body(nf_ref, w_ref, b_ref, o_ref):
    o_ref[...] = _softplus(
        jnp.dot(nf_ref[...], w_ref[...], preferred_element_type=jnp.float32)
        + b_ref[...]
    )


def _emb(nf, W, b_row):
    N, F = nf.shape
    B = 1000
    return pl.pallas_call(
        _emb_body,
        grid=(N // B,),
        in_specs=[
            pl.BlockSpec((B, F), lambda i: (i, 0)),
            pl.BlockSpec((F, F), lambda i: (0, 0)),
            pl.BlockSpec((1, F), lambda i: (0, 0)),
        ],
        out_specs=pl.BlockSpec((B, F), lambda i: (i, 0)),
        out_shape=jax.ShapeDtypeStruct((N, F), jnp.float32),
    )(nf, W, b_row)


_NBUF = 5


def _gather_rows(table, idx3):
    NW, nch, CH = idx3.shape
    F = table.shape[1]
    per = nch * CH
    NEp = NW * per
    ngrp = nch // _NBUF
    mesh = plsc.VectorSubcoreMesh(core_axis_name="c", subcore_axis_name="s")

    @functools.partial(
        pl.kernel,
        out_type=jax.ShapeDtypeStruct((NEp, F), table.dtype),
        mesh=mesh,
        scratch_types=[
            pltpu.VMEM((nch, CH), jnp.int32),
            pltpu.VMEM((_NBUF, CH, F), table.dtype),
        ]
        + [pltpu.SemaphoreType.DMA] * (2 * _NBUF),
    )
    def gk(table_hbm, idx_hbm, out_hbm, idx_v, rows_v, *sems):
        sem_g = sems[:_NBUF]
        sem_s = sems[_NBUF:]
        wid = lax.axis_index("s") * 2 + lax.axis_index("c")
        base = wid * per
        pltpu.sync_copy(idx_hbm.at[wid], idx_v)

        def wait_scatter(b):
            pltpu.make_async_copy(
                rows_v.at[b], out_hbm.at[pl.ds(0, CH)], sem_s[b]
            ).wait()

        def step(g, carry):
            handles = []
            for b in range(_NBUF):

                @pl.when(g > 0)
                def _():
                    wait_scatter(b)

                handles.append(
                    pltpu.async_copy(
                        table_hbm.at[idx_v.at[g * _NBUF + b]],
                        rows_v.at[b],
                        sem_g[b],
                    )
                )
            for b in range(_NBUF):
                handles[b].wait()
                pltpu.async_copy(
                    rows_v.at[b],
                    out_hbm.at[pl.ds(base + (g * _NBUF + b) * CH, CH)],
                    sem_s[b],
                )
            return carry

        lax.fori_loop(0, ngrp, step, 0)
        for b in range(_NBUF):
            wait_scatter(b)

    return gk(table, idx3)


def _layer(x, nbr_rows, edge_flat, M, att_ws, msg_ws, B, off, G):
    N, F = x.shape
    E = edge_flat.shape[1]
    nh = len(msg_ws) // 3
    Nout = G * B

    def body(*refs):
        x_ref, nbr_ref, ed_ref = refs[0:3]
        wa_c, wa_n, wa_e, vsel = refs[3:7]
        w = refs[7 : 7 + nh * 3]
        xn_ref, al_ref = refs[7 + nh * 3 :]
        xv = x_ref[...]
        nbr = nbr_ref[...].astype(jnp.float32)
        ed = ed_ref[...]

        def spl(v):
            return jnp.log2(1.0 + jnp.exp2(v))

        pre_a = jnp.dot(nbr, wa_n[...], preferred_element_type=jnp.float32)
        pre_a = pre_a + jnp.dot(ed, wa_e[...], preferred_element_type=jnp.float32)
        ca = jnp.dot(xv, wa_c[...], preferred_element_type=jnp.float32)
        KaT = pre_a.shape[1]
        spa = spl(pre_a.reshape(B, M, KaT) + ca[:, None, :]).reshape(B * M, KaT)
        e3 = jnp.dot(spa, vsel[...], preferred_element_type=jnp.float32).reshape(
            B, M, nh
        )
        wexp = jnp.exp(e3)
        alpha3 = wexp / jnp.sum(wexp, axis=1, keepdims=True)

        acc = None
        for h in range(nh):
            Wc_m, Wn_m, We_m = w[h * 3 : (h + 1) * 3]
            pre_m = jnp.dot(nbr, Wn_m[...], preferred_element_type=jnp.float32)
            pre_m = pre_m + jnp.dot(ed, We_m[...], preferred_element_type=jnp.float32)
            cm = jnp.dot(xv, Wc_m[...], preferred_element_type=jnp.float32)
            msg = spl(pre_m.reshape(B, M, F) + cm[:, None, :])
            out = jnp.sum(alpha3[:, :, h : h + 1] * msg, axis=1)
            acc = out if acc is None else acc + out
        xn_ref[...] = xv + acc * (_LN2 / nh)
        al_ref[...] = alpha3[:, :, 0]

    in_specs = [
        pl.BlockSpec((B, F), lambda i: (i + off, 0)),
        pl.BlockSpec((B * M, F), lambda i: (i, 0)),
        pl.BlockSpec((B * M, E), lambda i: (i + off, 0)),
    ]
    for wa in att_ws + msg_ws:
        s = wa.shape
        in_specs.append(pl.BlockSpec(s, lambda i: (0, 0)))
    return pl.pallas_call(
        body,
        grid=(G,),
        in_specs=in_specs,
        out_specs=[
            pl.BlockSpec((B, F), lambda i: (i, 0)),
            pl.BlockSpec((B, M), lambda i: (i, 0)),
        ],
        out_shape=[
            jax.ShapeDtypeStruct((Nout, F), jnp.float32),
            jax.ShapeDtypeStruct((Nout, M), jnp.float32),
        ],
    )(x, nbr_rows, edge_flat, *att_ws, *msg_ws)


def _pool_body(x_ref, wg_ref, wo_ref, bo_ref, o_ref):
    xv = x_ref[...]
    s = _softplus(jnp.sum(xv * wg_ref[...], axis=1, keepdims=True))
    mx = jnp.max(s)
    g = jnp.exp(s - mx)
    gf = jnp.sum(g * xv, axis=0, keepdims=True) / jnp.sum(g)
    z = jnp.sum(gf * wo_ref[...], axis=1)[None, :] + bo_ref[...]
    mz = jnp.max(z, axis=1, keepdims=True)
    o_ref[...] = z - (mz + jnp.log(jnp.sum(jnp.exp(z - mz), axis=1, keepdims=True)))


def _pool(x, Wg_row, Wo_t, bo_row):
    N, F = x.shape
    return pl.pallas_call(
        _pool_body,
        grid=(1,),
        in_specs=[
            pl.BlockSpec((N, F), lambda i: (0, 0)),
            pl.BlockSpec((1, F), lambda i: (0, 0)),
            pl.BlockSpec((2, F), lambda i: (0, 0)),
            pl.BlockSpec((1, 2), lambda i: (0, 0)),
        ],
        out_specs=pl.BlockSpec((1, 2), lambda i: (0, 0)),
        out_shape=jax.ShapeDtypeStruct((1, 2), jnp.float32),
    )(x, Wg_row, Wo_t, bo_row)


def _att_pack(heads, F):
    Was = [W * _LOG2E for W, _ in heads]
    Wa = jnp.concatenate(Was, axis=1) if len(Was) > 1 else Was[0]
    KaT = Wa.shape[1]
    cols = []
    off = 0
    for W, v in heads:
        k = W.shape[1]
        col = jnp.zeros((KaT,), jnp.float32).at[off : off + k].set(v * _LN2)
        cols.append(col)
        off += k
    vsel = jnp.stack(cols, axis=1)
    return [Wa[:F], Wa[F : 2 * F], Wa[2 * F :], vsel]


def _msg_ws(W_msg, F):
    Wm = W_msg * _LOG2E
    return [Wm[:F], Wm[F : 2 * F], Wm[2 * F :]]


def kernel(node_features, edge_features, neighbor_indices, neighbor_masks,
           W_emb, b_emb, W_att_0_0, v_att_0_0, W_msg_0_0, W_att_0_1, v_att_0_1,
           W_msg_0_1, W_att_1_0, v_att_1_0, W_msg_1_0, W_graph, W_out, b_out):
    N, F = node_features.shape
    M = neighbor_indices.shape[1]
    E = edge_features.shape[2]
    NE = N * M

    NW, CH = 32, 128
    half = NE // 2
    nch = -(-half // (NW * CH))
    padlen = NW * nch * CH - half
    idx_flat = neighbor_indices.astype(jnp.int32).reshape(NE)
    zpad = jnp.zeros((padlen,), jnp.int32)
    idxA = jnp.concatenate([idx_flat[:half], zpad]).reshape(NW, nch, CH)
    idxB = jnp.concatenate([idx_flat[half:], zpad]).reshape(NW, nch, CH)
    edge_flat = edge_features.reshape(NE, E)

    B = 200
    G = (N // 2) // B

    x0 = _emb(node_features, W_emb, b_emb.reshape(1, F))

    att0 = _att_pack([(W_att_0_0, v_att_0_0), (W_att_0_1, v_att_0_1)], F)
    msg0 = _msg_ws(W_msg_0_0, F) + _msg_ws(W_msg_0_1, F)
    g0a = _gather_rows(x0, idxA)
    g0b = _gather_rows(x0, idxB)
    x1a, _ = _layer(x0, g0a, edge_flat, M, att0, msg0, B, 0, G)
    x1b, _ = _layer(x0, g0b, edge_flat, M, att0, msg0, B, G, G)
    x1 = jnp.concatenate([x1a, x1b])

    att1 = _att_pack([(W_att_1_0, v_att_1_0)], F)
    msg1 = _msg_ws(W_msg_1_0, F)
    g1a = _gather_rows(x1, idxA)
    g1b = _gather_rows(x1, idxB)
    x2a, alpha_a = _layer(x1, g1a, edge_flat, M, att1, msg1, B, 0, G)
    x2b, alpha_b = _layer(x1, g1b, edge_flat, M, att1, msg1, B, G, G)
    x2 = jnp.concatenate([x2a, x2b])
    alpha = jnp.concatenate([alpha_a, alpha_b])

    preds = _pool(x2, W_graph.reshape(1, F), W_out.T, b_out.reshape(1, 2))
    return preds, alpha


# speedup vs baseline: 1.5916x; 1.5916x over previous
import functools

import jax
import jax.numpy as jnp
from jax import lax
from jax.experimental import pallas as pl
from jax.experimental.pallas import tpu as pltpu
from jax.experimental.pallas import tpu_sc as plsc


_LOG2E = 1.4426950408889634
_LN2 = 0.6931471805599453


def _softplus(v):
    return jnp.log2(1.0 + jnp.exp2(v * _LOG2E)) * _LN2


def _emb_body(nf_ref, w_ref, b_ref, o_ref):
    o_ref[...] = _softplus(
        jnp.dot(nf_ref[...], w_ref[...], preferred_element_type=jnp.float32)
        + b_ref[...]
    )


def _emb(nf, W, b_row):
    N, F = nf.shape
    B = 1000
    return pl.pallas_call(
        _emb_body,
        grid=(N // B,),
        in_specs=[
            pl.BlockSpec((B, F), lambda i: (i, 0)),
            pl.BlockSpec((F, F), lambda i: (0, 0)),
            pl.BlockSpec((1, F), lambda i: (0, 0)),
        ],
        out_specs=pl.BlockSpec((B, F), lambda i: (i, 0)),
        out_shape=jax.ShapeDtypeStruct((N, F), jnp.float32),
    )(nf, W, b_row)


_NBUF = 5


def _gather_rows(table, idx3):
    NW, nch, CH = idx3.shape
    F = table.shape[1]
    per = nch * CH
    NEp = NW * per
    ngrp = nch // _NBUF
    mesh = plsc.VectorSubcoreMesh(core_axis_name="c", subcore_axis_name="s")

    @functools.partial(
        pl.kernel,
        out_type=jax.ShapeDtypeStruct((NEp, F), table.dtype),
        mesh=mesh,
        scratch_types=[
            pltpu.VMEM((nch, CH), jnp.int32),
            pltpu.VMEM((_NBUF, CH, F), table.dtype),
        ]
        + [pltpu.SemaphoreType.DMA] * (2 * _NBUF),
    )
    def gk(table_hbm, idx_hbm, out_hbm, idx_v, rows_v, *sems):
        sem_g = sems[:_NBUF]
        sem_s = sems[_NBUF:]
        wid = lax.axis_index("s") * 2 + lax.axis_index("c")
        base = wid * per
        pltpu.sync_copy(idx_hbm.at[wid], idx_v)

        def wait_scatter(b):
            pltpu.make_async_copy(
                rows_v.at[b], out_hbm.at[pl.ds(0, CH)], sem_s[b]
            ).wait()

        def step(g, carry):
            handles = []
            for b in range(_NBUF):

                @pl.when(g > 0)
                def _():
                    wait_scatter(b)

                handles.append(
                    pltpu.async_copy(
                        table_hbm.at[idx_v.at[g * _NBUF + b]],
                        rows_v.at[b],
                        sem_g[b],
                    )
                )
            for b in range(_NBUF):
                handles[b].wait()
                pltpu.async_copy(
                    rows_v.at[b],
                    out_hbm.at[pl.ds(base + (g * _NBUF + b) * CH, CH)],
                    sem_s[b],
                )
            return carry

        lax.fori_loop(0, ngrp, step, 0)
        for b in range(_NBUF):
            wait_scatter(b)

    return gk(table, idx3)


def _layer(x, nbr_rows, edge_flat, M, att_ws, msg_ws, B, off, G):
    N, F = x.shape
    E = edge_flat.shape[1]
    nh = len(msg_ws) // 3
    Nout = G * B

    def body(*refs):
        x_ref, nbr_ref, ed_ref = refs[0:3]
        wa_c, wa_n, wa_e, vsel = refs[3:7]
        w = refs[7 : 7 + nh * 3]
        xn_ref, al_ref = refs[7 + nh * 3 :]
        xv = x_ref[...]
        nbr = nbr_ref[...].astype(jnp.float32)
        ed = ed_ref[...]

        def spl(v):
            return jnp.log2(1.0 + jnp.exp2(v))

        pre_a = jnp.dot(nbr, wa_n[...], preferred_element_type=jnp.float32)
        pre_a = pre_a + jnp.dot(ed, wa_e[...], preferred_element_type=jnp.float32)
        ca = jnp.dot(xv, wa_c[...], preferred_element_type=jnp.float32)
        KaT = pre_a.shape[1]
        spa = spl(pre_a.reshape(B, M, KaT) + ca[:, None, :]).reshape(B * M, KaT)
        e3 = jnp.dot(spa, vsel[...], preferred_element_type=jnp.float32).reshape(
            B, M, nh
        )
        wexp = jnp.exp(e3)
        alpha3 = wexp / jnp.sum(wexp, axis=1, keepdims=True)

        acc = None
        for h in range(nh):
            Wc_m, Wn_m, We_m = w[h * 3 : (h + 1) * 3]
            pre_m = jnp.dot(nbr, Wn_m[...], preferred_element_type=jnp.float32)
            pre_m = pre_m + jnp.dot(ed, We_m[...], preferred_element_type=jnp.float32)
            cm = jnp.dot(xv, Wc_m[...], preferred_element_type=jnp.float32)
            msg = spl(pre_m.reshape(B, M, F) + cm[:, None, :])
            out = jnp.sum(alpha3[:, :, h : h + 1] * msg, axis=1)
            acc = out if acc is None else acc + out
        xn_ref[...] = xv + acc * (_LN2 / nh)
        al_ref[...] = alpha3[:, :, 0]

    in_specs = [
        pl.BlockSpec((B, F), lambda i: (i + off, 0)),
        pl.BlockSpec((B * M, F), lambda i: (i, 0)),
        pl.BlockSpec((B * M, E), lambda i: (i + off, 0)),
    ]
    for wa in att_ws + msg_ws:
        s = wa.shape
        in_specs.append(pl.BlockSpec(s, lambda i: (0, 0)))
    return pl.pallas_call(
        body,
        grid=(G,),
        in_specs=in_specs,
        out_specs=[
            pl.BlockSpec((B, F), lambda i: (i, 0)),
            pl.BlockSpec((B, M), lambda i: (i, 0)),
        ],
        out_shape=[
            jax.ShapeDtypeStruct((Nout, F), jnp.float32),
            jax.ShapeDtypeStruct((Nout, M), jnp.float32),
        ],
    )(x, nbr_rows, edge_flat, *att_ws, *msg_ws)


def _pool_body(x_ref, wg_ref, wo_ref, bo_ref, o_ref):
    xv = x_ref[...]
    s = _softplus(jnp.sum(xv * wg_ref[...], axis=1, keepdims=True))
    mx = jnp.max(s)
    g = jnp.exp(s - mx)
    gf = jnp.sum(g * xv, axis=0, keepdims=True) / jnp.sum(g)
    z = jnp.sum(gf * wo_ref[...], axis=1)[None, :] + bo_ref[...]
    mz = jnp.max(z, axis=1, keepdims=True)
    o_ref[...] = z - (mz + jnp.log(jnp.sum(jnp.exp(z - mz), axis=1, keepdims=True)))


def _pool(x, Wg_row, Wo_t, bo_row):
    N, F = x.shape
    return pl.pallas_call(
        _pool_body,
        grid=(1,),
        in_specs=[
            pl.BlockSpec((N, F), lambda i: (0, 0)),
            pl.BlockSpec((1, F), lambda i: (0, 0)),
            pl.BlockSpec((2, F), lambda i: (0, 0)),
            pl.BlockSpec((1, 2), lambda i: (0, 0)),
        ],
        out_specs=pl.BlockSpec((1, 2), lambda i: (0, 0)),
        out_shape=jax.ShapeDtypeStruct((1, 2), jnp.float32),
    )(x, Wg_row, Wo_t, bo_row)


def _att_pack(heads, F):
    Was = [W * _LOG2E for W, _ in heads]
    Wa = jnp.concatenate(Was, axis=1) if len(Was) > 1 else Was[0]
    KaT = Wa.shape[1]
    cols = []
    off = 0
    for W, v in heads:
        k = W.shape[1]
        col = jnp.zeros((KaT,), jnp.float32).at[off : off + k].set(v * _LN2)
        cols.append(col)
        off += k
    vsel = jnp.stack(cols, axis=1)
    return [Wa[:F], Wa[F : 2 * F], Wa[2 * F :], vsel]


def _msg_ws(W_msg, F):
    Wm = W_msg * _LOG2E
    return [Wm[:F], Wm[F : 2 * F], Wm[2 * F :]]


def kernel(node_features, edge_features, neighbor_indices, neighbor_masks,
           W_emb, b_emb, W_att_0_0, v_att_0_0, W_msg_0_0, W_att_0_1, v_att_0_1,
           W_msg_0_1, W_att_1_0, v_att_1_0, W_msg_1_0, W_graph, W_out, b_out):
    N, F = node_features.shape
    M = neighbor_indices.shape[1]
    E = edge_features.shape[2]
    NE = N * M

    NW, CH = 32, 80
    idx3 = neighbor_indices.astype(jnp.int32).reshape(NW, NE // (NW * CH), CH)
    edge_flat = edge_features.reshape(NE, E)

    B = 200
    G = N // B

    x0 = _emb(node_features, W_emb, b_emb.reshape(1, F))

    att0 = _att_pack([(W_att_0_0, v_att_0_0), (W_att_0_1, v_att_0_1)], F)
    msg0 = _msg_ws(W_msg_0_0, F) + _msg_ws(W_msg_0_1, F)
    g0 = _gather_rows(x0, idx3)
    x1, _ = _layer(x0, g0, edge_flat, M, att0, msg0, B, 0, G)

    att1 = _att_pack([(W_att_1_0, v_att_1_0)], F)
    msg1 = _msg_ws(W_msg_1_0, F)
    g1 = _gather_rows(x1, idx3)
    x2, alpha = _layer(x1, g1, edge_flat, M, att1, msg1, B, 0, G)

    preds = _pool(x2, W_graph.reshape(1, F), W_out.T, b_out.reshape(1, 2))
    return preds, alpha
